# Initial kernel scaffold; baseline (speedup 1.0000x reference)
import jax
import jax.numpy as jnp
from jax.experimental import pallas as pl


def kernel(selected_indices, pattern_weights, firing_patterns, W2_w, W2_b):
    raise NotImplementedError("write your pallas kernel here")


# speedup vs baseline: 3.1385x; 3.1385x over previous
import functools

import jax
import jax.numpy as jnp
from jax import lax
from jax.experimental import pallas as pl
from jax.experimental.pallas import tpu as pltpu
from jax.experimental.pallas import tpu_sc as plsc

POOL = 16384
DFF = 3072
DM = 768
NTOK = 8192
K = 8
NC, NS, LANES = 2, 16, 16
NW = NC * NS
TPW = NTOK // NW
GRP = TPW // LANES
OUT_T = 8
UNROLL = 2

_f32 = jnp.float32
_i32 = jnp.int32


def _sc_body(fp_hbm, idx_hbm, w_hbm, out_hbm,
             idx_v, w_v, rows_v, out_v, gsem0, gsem1, osem0, osem1):
    wid = lax.axis_index("s") * NC + lax.axis_index("c")
    base = wid * TPW

    pltpu.sync_copy(idx_hbm.at[pl.ds(base * K, TPW * K)], idx_v)
    pltpu.sync_copy(w_hbm.at[:, pl.ds(base, TPW)], w_v)

    def softmax_g(g, carry):
        col = g * LANES
        wv = [w_v[k, pl.ds(col, LANES)] for k in range(K)]
        m = wv[0]
        for k in range(1, K):
            m = jnp.maximum(m, wv[k])
        e = [jnp.exp(v - m) for v in wv]
        s = e[0]
        for k in range(1, K):
            s = s + e[k]
        inv = 1.0 / s
        for k in range(K):
            w_v[k, pl.ds(col, LANES)] = e[k] * inv
        return carry
    lax.fori_loop(0, GRP, softmax_g, 0)

    def g_copy(t, b):
        return pltpu.make_async_copy(
            fp_hbm.at[idx_v.at[pl.ds(t * K, K)]], rows_v.at[b],
            gsem0 if b == 0 else gsem1)

    def o_copy(row, ob):
        return pltpu.make_async_copy(
            out_v.at[ob], out_hbm.at[pl.ds(row, OUT_T)],
            osem0 if ob == 0 else osem1)

    g_copy(0, 0).start()
    g_copy(1, 1).start()

    def super_body(i, carry):
        wg = [w_v[k, pl.ds(i * LANES, LANES)] for k in range(K)]
        for j in range(16):
            t = i * 16 + j
            b = j % 2
            ob = j // 8
            if j == 0:
                @pl.when(i > 0)
                def _w0():
                    o_copy(base + (i - 1) * 16, 0).wait()
            if j == 8:
                @pl.when(i > 0)
                def _w1():
                    o_copy(base + (i - 1) * 16 + OUT_T, 1).wait()

            g_copy(t, b).wait()

            spl = [jnp.broadcast_to(wg[k][j], (LANES,)) for k in range(K)]

            def chunk_body(c, carry2, _b=b, _ob=ob, _jj=j % 8, _spl=spl):
                for u in range(UNROLL):
                    off = (c * UNROLL + u) * LANES
                    acc = _spl[0] * rows_v[_b, 0, pl.ds(off, LANES)]
                    for k in range(1, K):
                        acc = acc + _spl[k] * rows_v[_b, k, pl.ds(off, LANES)]
                    out_v[_ob, _jj, pl.ds(off, LANES)] = acc
                return carry2
            lax.fori_loop(0, DFF // (LANES * UNROLL), chunk_body, 0)

            @pl.when(t + 2 < TPW)
            def _g():
                g_copy(t + 2, b).start()

            if j == 7:
                o_copy(base + i * 16, 0).start()
            if j == 15:
                o_copy(base + i * 16 + OUT_T, 1).start()
        return carry

    lax.fori_loop(0, GRP, super_body, 0)

    o_copy(base + (GRP - 1) * 16, 0).wait()
    o_copy(base + (GRP - 1) * 16 + OUT_T, 1).wait()


_sc_combine = functools.partial(
    pl.kernel,
    out_type=jax.ShapeDtypeStruct((NTOK, DFF), _f32),
    mesh=plsc.VectorSubcoreMesh(
        core_axis_name="c", subcore_axis_name="s",
        num_cores=NC, num_subcores=NS),
    scratch_types=[
        pltpu.VMEM((TPW * K,), _i32),
        pltpu.VMEM((K, TPW), _f32),
        pltpu.VMEM((2, K, DFF), _f32),
        pltpu.VMEM((2, OUT_T, DFF), _f32),
        pltpu.SemaphoreType.DMA,
        pltpu.SemaphoreType.DMA,
        pltpu.SemaphoreType.DMA,
        pltpu.SemaphoreType.DMA,
    ],
)(_sc_body)


TBLK = 512


def _tc_body(x_ref, w_ref, b_ref, o_ref):
    x = x_ref[...]
    a = 0.5 * x * (1.0 + lax.erf(x * (2.0 ** -0.5)))
    o_ref[...] = (jnp.dot(a, w_ref[...], preferred_element_type=_f32)
                  + b_ref[...])


def _tc_gelu_matmul(x, wt, b2):
    return pl.pallas_call(
        _tc_body,
        grid=(NTOK // TBLK,),
        in_specs=[
            pl.BlockSpec((TBLK, DFF), lambda i: (i, 0)),
            pl.BlockSpec((DFF, DM), lambda i: (0, 0)),
            pl.BlockSpec((1, DM), lambda i: (0, 0)),
        ],
        out_specs=pl.BlockSpec((TBLK, DM), lambda i: (i, 0)),
        out_shape=jax.ShapeDtypeStruct((NTOK, DM), _f32),
    )(x, wt, b2)


def kernel(selected_indices, pattern_weights, firing_patterns, W2_w, W2_b):
    B, S, _ = selected_indices.shape
    idx = selected_indices.reshape(NTOK * K).astype(_i32)
    wT = pattern_weights.reshape(NTOK, K).T
    combined = _sc_combine(firing_patterns, idx, wT)
    out = _tc_gelu_matmul(combined, W2_w.T, W2_b.reshape(1, DM))
    return out.reshape(B, S, DM)
